# baseline (speedup 1.0000x reference)
import functools

import jax
import jax.numpy as jnp
from jax.experimental import pallas as pl
from jax.experimental.pallas import tpu as pltpu

_NC = 4


def _round_up(x, m):
    return ((x + m - 1) // m) * m


def _lstm_fused_kernel(ids_ref, tab_ref, wcat_ref, bias_ref,
                       wout_ref, bout_ref, out_ref, *scr):
    xg = [scr[2 * i:2 * i + 2] for i in range(_NC)]
    h_scr = scr[2 * _NC:3 * _NC]
    c_scr = scr[3 * _NC:4 * _NC]
    m_scr = scr[4 * _NC:5 * _NC]

    t_blk_idx = pl.program_id(0)
    Hp = c_scr[0].shape[1]
    Bb = c_scr[0].shape[0]
    Bp = _NC * Bb
    T_BLK = (ids_ref.shape[0] // Bp) // pl.num_programs(0)

    @pl.when(t_blk_idx == 0)
    def _init():
        for scr_ in h_scr + c_scr:
            scr_[...] = jnp.zeros_like(scr_)
        for scr_ in m_scr:
            scr_[...] = jnp.full(scr_.shape, -jnp.inf, scr_.dtype)

    wcat = wcat_ref[...]
    bias = bias_ref[...]

    def step(t, ci, h_bf):
        xgb = xg[ci][t % 2]
        base = t_blk_idx * (T_BLK * Bp) + t * Bp + ci * Bb
        for r in range(Bb):
            idx = ids_ref[base + r]
            xgb[pl.ds(r, 1), :] = tab_ref[pl.ds(idx, 1), :]
        hx = jnp.concatenate([h_bf, xgb[...].astype(jnp.bfloat16)], axis=1)
        gp = jnp.dot(hx, wcat, preferred_element_type=jnp.float32) + bias
        ti = jnp.tanh(gp[:, 0 * Hp:1 * Hp])
        tf = jnp.tanh(gp[:, 1 * Hp:2 * Hp])
        tg = jnp.tanh(gp[:, 2 * Hp:3 * Hp])
        to = jnp.tanh(gp[:, 3 * Hp:4 * Hp])
        c = c_scr[ci][...]
        c_new = 0.5 * ((c + tg) + (tf * c + ti * tg))
        tc = jnp.tanh(c_new)
        h_new = 0.5 * (tc + to * tc)
        c_scr[ci][...] = c_new
        m_scr[ci][...] = jnp.maximum(m_scr[ci][...], h_new)
        return h_new.astype(jnp.bfloat16)

    h = [h_scr[ci][...] for ci in range(_NC)]
    for t in range(T_BLK):
        for ci in range(_NC):
            h[ci] = step(t, ci, h[ci])
    for ci in range(_NC):
        h_scr[ci][...] = h[ci]

    @pl.when(t_blk_idx == pl.num_programs(0) - 1)
    def _finalize():
        wout = wout_ref[...]
        for ci in range(_NC):
            out_ref[ci * Bb:(ci + 1) * Bb] = (
                jnp.dot(m_scr[ci][...].astype(wout.dtype), wout,
                        preferred_element_type=jnp.float32)
                + bout_ref[...])


@functools.partial(jax.jit, static_argnames=("t_blk",))
def _forward(input_seq, emb_table, w_ih, w_hh, b_ih, b_hh, w_out, b_out,
             *, t_blk=32):
    B, T = input_seq.shape
    V, E = emb_table.shape
    H = w_hh.shape[1]
    C = w_out.shape[0]

    Bp = _round_up(B, 8 * _NC)
    Bb = Bp // _NC
    Hp = _round_up(H, 128)
    Cp = _round_up(C, 128)

    def pad_gate_cols(w_t):
        k = w_t.shape[0]
        w4 = w_t.reshape(k, 4, H)
        w4 = jnp.pad(w4, ((0, 0), (0, 0), (0, Hp - H)))
        return w4.reshape(k, 4 * Hp)

    gate_scale = jnp.repeat(jnp.array([0.5, 0.5, 1.0, 0.5], jnp.float32), Hp)
    whh_p = jnp.pad(pad_gate_cols(w_hh.T), ((0, Hp - H), (0, 0)))
    wih_p = pad_gate_cols(w_ih.T)
    wcat = (jnp.concatenate([whh_p, wih_p], axis=0)
            * gate_scale[None, :]).astype(jnp.bfloat16)
    bias = jnp.pad((b_ih + b_hh).astype(jnp.float32).reshape(4, H),
                   ((0, 0), (0, Hp - H))).reshape(1, 4 * Hp) * gate_scale[None, :]
    wout_p = jnp.pad(w_out.T, ((0, Hp - H), (0, Cp - C))).astype(jnp.bfloat16)
    bout_p = jnp.pad(b_out.astype(jnp.float32), (0, Cp - C)).reshape(1, Cp)

    ids = input_seq.T
    if Bp != B:
        ids = jnp.pad(ids, ((0, 0), (0, Bp - B)))
    ids = ids.reshape(T * Bp)

    while T % t_blk:
        t_blk //= 2
    grid = (T // t_blk,)

    per_chain_scr = []
    for _ in range(_NC):
        per_chain_scr += [pltpu.VMEM((Bb, E), jnp.float32)] * 2
    per_chain_scr += [pltpu.VMEM((Bb, Hp), jnp.bfloat16)] * _NC
    per_chain_scr += [pltpu.VMEM((Bb, Hp), jnp.float32)] * _NC
    per_chain_scr += [pltpu.VMEM((Bb, Hp), jnp.float32)] * _NC

    out = pl.pallas_call(
        _lstm_fused_kernel,
        out_shape=jax.ShapeDtypeStruct((Bp, Cp), jnp.float32),
        grid_spec=pltpu.PrefetchScalarGridSpec(
            num_scalar_prefetch=1,
            grid=grid,
            in_specs=[
                pl.BlockSpec((V, E), lambda t, ids_r: (0, 0)),
                pl.BlockSpec((Hp + E, 4 * Hp), lambda t, ids_r: (0, 0)),
                pl.BlockSpec((1, 4 * Hp), lambda t, ids_r: (0, 0)),
                pl.BlockSpec((Hp, Cp), lambda t, ids_r: (0, 0)),
                pl.BlockSpec((1, Cp), lambda t, ids_r: (0, 0)),
            ],
            out_specs=pl.BlockSpec((Bp, Cp), lambda t, ids_r: (0, 0)),
            scratch_shapes=per_chain_scr,
        ),
        compiler_params=pltpu.CompilerParams(
            dimension_semantics=("arbitrary",),
            vmem_limit_bytes=100 * 1024 * 1024,
        ),
    )(ids, emb_table, wcat, bias, wout_p, bout_p)

    return out[:B, :C]


def kernel(input_seq, emb_table, w_ih, w_hh, b_ih, b_hh, w_out, b_out):
    return _forward(input_seq, emb_table, w_ih, w_hh, b_ih, b_hh,
                    w_out, b_out, t_blk=32)


# speedup vs baseline: 1.4564x; 1.4564x over previous
import functools

import jax
import jax.numpy as jnp
from jax.experimental import pallas as pl
from jax.experimental.pallas import tpu as pltpu

_NC = 2


def _round_up(x, m):
    return ((x + m - 1) // m) * m


def _lstm_fused_kernel(ids_ref, tab_ref, wcat_ref, bias_ref,
                       wout_ref, bout_ref, out_ref, *scr):
    xg = [scr[2 * i:2 * i + 2] for i in range(_NC)]
    h_scr = scr[2 * _NC:3 * _NC]
    c_scr = scr[3 * _NC:4 * _NC]
    m_scr = scr[4 * _NC:5 * _NC]

    t_blk_idx = pl.program_id(0)
    Hp = c_scr[0].shape[1]
    Bb = c_scr[0].shape[0]
    Bp = _NC * Bb
    T_BLK = (ids_ref.shape[0] // Bp) // pl.num_programs(0)

    @pl.when(t_blk_idx == 0)
    def _init():
        for scr_ in h_scr + c_scr:
            scr_[...] = jnp.zeros_like(scr_)
        for scr_ in m_scr:
            scr_[...] = jnp.full(scr_.shape, -jnp.inf, scr_.dtype)

    wcat = wcat_ref[...]
    bias = bias_ref[...]

    def step(t, ci, h_bf):
        xgb = xg[ci][t % 2]
        base = t_blk_idx * (T_BLK * Bp) + t * Bp + ci * Bb
        for r in range(Bb):
            idx = ids_ref[base + r]
            xgb[pl.ds(r, 1), :] = tab_ref[pl.ds(idx, 1), :]
        hx = jnp.concatenate([h_bf, xgb[...].astype(jnp.bfloat16)], axis=1)
        gp = jnp.dot(hx, wcat, preferred_element_type=jnp.float32) + bias
        ti = jnp.tanh(gp[:, 0 * Hp:1 * Hp])
        tf = jnp.tanh(gp[:, 1 * Hp:2 * Hp])
        tg = jnp.tanh(gp[:, 2 * Hp:3 * Hp])
        to = jnp.tanh(gp[:, 3 * Hp:4 * Hp])
        c = c_scr[ci][...]
        c_new = 0.5 * ((c + tg) + (tf * c + ti * tg))
        tc = jnp.tanh(c_new)
        h_new = 0.5 * (tc + to * tc)
        c_scr[ci][...] = c_new
        m_scr[ci][...] = jnp.maximum(m_scr[ci][...], h_new)
        return h_new.astype(jnp.bfloat16)

    h = [h_scr[ci][...] for ci in range(_NC)]
    for t in range(T_BLK):
        for ci in range(_NC):
            h[ci] = step(t, ci, h[ci])
    for ci in range(_NC):
        h_scr[ci][...] = h[ci]

    @pl.when(t_blk_idx == pl.num_programs(0) - 1)
    def _finalize():
        wout = wout_ref[...]
        for ci in range(_NC):
            out_ref[ci * Bb:(ci + 1) * Bb] = (
                jnp.dot(m_scr[ci][...].astype(wout.dtype), wout,
                        preferred_element_type=jnp.float32)
                + bout_ref[...])


@functools.partial(jax.jit, static_argnames=("t_blk",))
def _forward(input_seq, emb_table, w_ih, w_hh, b_ih, b_hh, w_out, b_out,
             *, t_blk=32):
    B, T = input_seq.shape
    V, E = emb_table.shape
    H = w_hh.shape[1]
    C = w_out.shape[0]

    Bp = _round_up(B, 8 * _NC)
    Bb = Bp // _NC
    Hp = _round_up(H, 128)
    Cp = _round_up(C, 128)

    def pad_gate_cols(w_t):
        k = w_t.shape[0]
        w4 = w_t.reshape(k, 4, H)
        w4 = jnp.pad(w4, ((0, 0), (0, 0), (0, Hp - H)))
        return w4.reshape(k, 4 * Hp)

    gate_scale = jnp.repeat(jnp.array([0.5, 0.5, 1.0, 0.5], jnp.float32), Hp)
    whh_p = jnp.pad(pad_gate_cols(w_hh.T), ((0, Hp - H), (0, 0)))
    wih_p = pad_gate_cols(w_ih.T)
    wcat = (jnp.concatenate([whh_p, wih_p], axis=0)
            * gate_scale[None, :]).astype(jnp.bfloat16)
    bias = jnp.pad((b_ih + b_hh).astype(jnp.float32).reshape(4, H),
                   ((0, 0), (0, Hp - H))).reshape(1, 4 * Hp) * gate_scale[None, :]
    wout_p = jnp.pad(w_out.T, ((0, Hp - H), (0, Cp - C))).astype(jnp.bfloat16)
    bout_p = jnp.pad(b_out.astype(jnp.float32), (0, Cp - C)).reshape(1, Cp)

    ids = input_seq.T
    if Bp != B:
        ids = jnp.pad(ids, ((0, 0), (0, Bp - B)))
    ids = ids.reshape(T * Bp)

    while T % t_blk:
        t_blk //= 2
    grid = (T // t_blk,)

    per_chain_scr = []
    for _ in range(_NC):
        per_chain_scr += [pltpu.VMEM((Bb, E), jnp.float32)] * 2
    per_chain_scr += [pltpu.VMEM((Bb, Hp), jnp.bfloat16)] * _NC
    per_chain_scr += [pltpu.VMEM((Bb, Hp), jnp.float32)] * _NC
    per_chain_scr += [pltpu.VMEM((Bb, Hp), jnp.float32)] * _NC

    out = pl.pallas_call(
        _lstm_fused_kernel,
        out_shape=jax.ShapeDtypeStruct((Bp, Cp), jnp.float32),
        grid_spec=pltpu.PrefetchScalarGridSpec(
            num_scalar_prefetch=1,
            grid=grid,
            in_specs=[
                pl.BlockSpec((V, E), lambda t, ids_r: (0, 0)),
                pl.BlockSpec((Hp + E, 4 * Hp), lambda t, ids_r: (0, 0)),
                pl.BlockSpec((1, 4 * Hp), lambda t, ids_r: (0, 0)),
                pl.BlockSpec((Hp, Cp), lambda t, ids_r: (0, 0)),
                pl.BlockSpec((1, Cp), lambda t, ids_r: (0, 0)),
            ],
            out_specs=pl.BlockSpec((Bp, Cp), lambda t, ids_r: (0, 0)),
            scratch_shapes=per_chain_scr,
        ),
        compiler_params=pltpu.CompilerParams(
            dimension_semantics=("arbitrary",),
            vmem_limit_bytes=100 * 1024 * 1024,
        ),
    )(ids, emb_table, wcat, bias, wout_p, bout_p)

    return out[:B, :C]


def kernel(input_seq, emb_table, w_ih, w_hh, b_ih, b_hh, w_out, b_out):
    return _forward(input_seq, emb_table, w_ih, w_hh, b_ih, b_hh,
                    w_out, b_out, t_blk=32)
